# baseline (speedup 1.0000x reference)
import functools

import jax
import jax.numpy as jnp
from jax import lax
from jax.experimental import pallas as pl
from jax.experimental.pallas import tpu as pltpu
from jax.experimental.pallas import tpu_sc as plsc

N = 10000
E = 320000
D = 128
DH = D // 2

NC = 2
NS = 16
NW = NC * NS
L = 16

NPAD = 10240
RPT = NPAD // NS
K = 128
DEGW = 16

EPT1 = E // NW
CH1 = 80
EPAD1 = CH1 * K

EPT2 = E // NS
KE = 128
CH2 = 157
EPAD2 = CH2 * KE

GRP = 5
NGRP = CH2 // GRP

_MESH = plsc.VectorSubcoreMesh(core_axis_name="c", subcore_axis_name="s",
                               num_cores=NC, num_subcores=NS)
_SC_PARAMS = pltpu.CompilerParams(use_tc_tiling_on_sc=False)


def _deg_body(dst_hbm, zdeg_hbm, out_hbm, idx_v, ones_v, acc_sh):
    c = lax.axis_index("c")
    s = lax.axis_index("s")
    w = c * NS + s
    pltpu.sync_copy(dst_hbm.at[w], idx_v)
    pltpu.sync_copy(zdeg_hbm.at[pl.ds(s * RPT, RPT)],
                    acc_sh.at[pl.ds(s * RPT, RPT)])
    one = jnp.full((L,), 1.0, dtype=jnp.float32)

    def fill(i, carry):
        ones_v[i, :] = one
        return carry

    lax.fori_loop(0, K, fill, 0)
    plsc.subcore_barrier()

    def chunk(jc, carry):
        pltpu.sync_copy(ones_v, acc_sh.at[idx_v.at[jc]], add=True)
        return carry

    lax.fori_loop(0, CH1, chunk, 0)
    plsc.subcore_barrier()
    pltpu.sync_copy(acc_sh.at[pl.ds(s * RPT, RPT)],
                    out_hbm.at[c, pl.ds(s * RPT, RPT)])


_deg_kernel = functools.partial(
    pl.kernel,
    out_type=jax.ShapeDtypeStruct((NC, NPAD, DEGW), jnp.float32),
    mesh=_MESH,
    scratch_types=[
        pltpu.VMEM((CH1, K), jnp.int32),
        pltpu.VMEM((K, DEGW), jnp.float32),
        pltpu.VMEM_SHARED((NPAD, DEGW), jnp.float32),
    ],
    compiler_params=_SC_PARAMS,
)(_deg_body)


def _edge_body(src_hbm, dst_hbm, y2_hbm, zy_hbm, out_hbm,
               srcv, dstv, bufs, acc_sh, gsem, ssem):
    c = lax.axis_index("c")
    s = lax.axis_index("s")
    pltpu.sync_copy(src_hbm.at[c, s], srcv)
    pltpu.sync_copy(dst_hbm.at[s], dstv)
    pltpu.sync_copy(zy_hbm.at[pl.ds(s * RPT, RPT)],
                    acc_sh.at[pl.ds(s * RPT, RPT)])
    plsc.subcore_barrier()
    ytab = y2_hbm

    def gather(jc, slot):
        return pltpu.make_async_copy(ytab.at[srcv.at[jc]], bufs.at[slot],
                                     gsem.at[slot])

    def scatter(jc, slot):
        return pltpu.make_async_copy(bufs.at[slot], acc_sh.at[dstv.at[jc]],
                                     ssem.at[slot])

    for j in range(GRP):
        gather(j, j).start()

    def group(g, carry):
        base = g * GRP
        for b in (0, 1, 2):
            gather(base + b, b).wait()
            scatter(base + b, b).start(add=True)
        for b in (3, 4):
            gather(base + b, b).wait()
            scatter(base + b, b).start(add=True)
        for b in (0, 1, 2):
            scatter(base + b, b).wait()

            @pl.when(base + GRP + b < NGRP * GRP)
            def _(b=b):
                gather(base + GRP + b, b).start()
        for b in (3, 4):
            scatter(base + b, b).wait()

            @pl.when(base + GRP + b < NGRP * GRP)
            def _(b=b):
                gather(base + GRP + b, b).start()
        return carry

    lax.fori_loop(0, NGRP, group, 0)
    for jc in range(NGRP * GRP, CH2):
        gather(jc, 0).start()
        gather(jc, 0).wait()
        scatter(jc, 0).start(add=True)
        scatter(jc, 0).wait()
    plsc.subcore_barrier()
    pltpu.sync_copy(acc_sh.at[pl.ds(s * RPT, RPT)],
                    out_hbm.at[c, pl.ds(s * RPT, RPT)])


_edge_kernel = functools.partial(
    pl.kernel,
    out_type=jax.ShapeDtypeStruct((NC, NPAD, DH), jnp.float32),
    mesh=_MESH,
    scratch_types=[
        pltpu.VMEM((CH2, KE), jnp.int32),
        pltpu.VMEM((CH2, KE), jnp.int32),
        pltpu.VMEM((GRP, KE, DH), jnp.float32),
        pltpu.VMEM_SHARED((NPAD, DH), jnp.float32),
        pltpu.SemaphoreType.DMA((GRP,)),
        pltpu.SemaphoreType.DMA((GRP,)),
    ],
    compiler_params=_SC_PARAMS,
)(_edge_body)


BR2 = 2000


def _mm_body(x_ref, w_ref, dg_ref, y_ref):
    deg = dg_ref[0, :, 0:1] + dg_ref[1, :, 0:1] + 1.0
    dinv = lax.rsqrt(deg)
    xw = jnp.dot(x_ref[...], w_ref[...], preferred_element_type=jnp.float32)
    y_ref[...] = xw * dinv


def _mm_kernel(x, w, dega):
    return pl.pallas_call(
        _mm_body,
        grid=(N // BR2,),
        in_specs=[
            pl.BlockSpec((BR2, D), lambda i: (i, 0)),
            pl.BlockSpec((D, D), lambda i: (0, 0)),
            pl.BlockSpec((NC, BR2, DEGW), lambda i: (0, i, 0)),
        ],
        out_specs=pl.BlockSpec((BR2, D), lambda i: (i, 0)),
        out_shape=jax.ShapeDtypeStruct((N, D), jnp.float32),
    )(x, w, dega)


BR4 = 1000


def _ep_body(acc_ref, y_ref, dg_ref, b_ref, o_ref):
    deg = dg_ref[0, :, 0:1] + dg_ref[1, :, 0:1] + 1.0
    dinv = lax.rsqrt(deg)
    h = jnp.concatenate([acc_ref[0], acc_ref[1]], axis=1) + y_ref[...]
    o_ref[...] = h * dinv + b_ref[...]


def _ep_kernel(acc, y, dega, b2):
    return pl.pallas_call(
        _ep_body,
        grid=(N // BR4,),
        in_specs=[
            pl.BlockSpec((NC, BR4, DH), lambda i: (0, i, 0)),
            pl.BlockSpec((BR4, D), lambda i: (i, 0)),
            pl.BlockSpec((NC, BR4, DEGW), lambda i: (0, i, 0)),
            pl.BlockSpec((1, D), lambda i: (0, 0)),
        ],
        out_specs=pl.BlockSpec((BR4, D), lambda i: (i, 0)),
        out_shape=jax.ShapeDtypeStruct((N, D), jnp.float32),
    )(acc, y, dega, b2)


def kernel(x, edge_index, W, b):
    src = edge_index[0]
    dst = edge_index[1]
    pad1 = N + jnp.arange(EPAD1 - EPT1, dtype=jnp.int32) % (NPAD - N)
    pad2 = N + jnp.arange(EPAD2 - EPT2, dtype=jnp.int32) % (NPAD - N)
    dst_t1 = jnp.concatenate(
        [dst.reshape(NW, EPT1),
         jnp.broadcast_to(pad1, (NW, EPAD1 - EPT1))],
        axis=1).reshape(NW, CH1, K)
    src2 = src * 2
    src_t2 = jnp.pad(jnp.stack([src2, src2 + 1]).reshape(NC, NS, EPT2),
                     ((0, 0), (0, 0), (0, EPAD2 - EPT2)),
                     constant_values=0).reshape(NC, NS, CH2, KE)
    dst_t2 = jnp.concatenate(
        [dst.reshape(NS, EPT2),
         jnp.broadcast_to(pad2, (NS, EPAD2 - EPT2))],
        axis=1).reshape(NS, CH2, KE)
    zeros_deg = jnp.zeros((NPAD, DEGW), jnp.float32)
    zeros_y = jnp.zeros((NPAD, DH), jnp.float32)

    dega = _deg_kernel(dst_t1, zeros_deg)
    y = _mm_kernel(x, W, dega)
    acc = _edge_kernel(src_t2, dst_t2, y.reshape(2 * N, DH), zeros_y)
    return _ep_kernel(acc, y, dega, b.reshape(1, D))


# speedup vs baseline: 1.0100x; 1.0100x over previous
import functools

import jax
import jax.numpy as jnp
from jax import lax
from jax.experimental import pallas as pl
from jax.experimental.pallas import tpu as pltpu
from jax.experimental.pallas import tpu_sc as plsc

N = 10000
E = 320000
D = 128
DH = D // 2

NC = 2
NS = 16
NW = NC * NS
L = 16

NPAD = 10240
RPT = NPAD // NS
K = 128
DEGW = 16

EPT1 = E // NW
CH1 = 80
EPAD1 = CH1 * K

EPT2 = E // NS
KE = 128
CH2 = 157
EPAD2 = CH2 * KE

GRP = 6
NGRP = CH2 // GRP

_MESH = plsc.VectorSubcoreMesh(core_axis_name="c", subcore_axis_name="s",
                               num_cores=NC, num_subcores=NS)
_SC_PARAMS = pltpu.CompilerParams(use_tc_tiling_on_sc=False)


def _deg_body(dst_hbm, zdeg_hbm, out_hbm, idx_v, ones_v, acc_sh):
    c = lax.axis_index("c")
    s = lax.axis_index("s")
    w = c * NS + s
    pltpu.sync_copy(dst_hbm.at[w], idx_v)
    pltpu.sync_copy(zdeg_hbm.at[pl.ds(s * RPT, RPT)],
                    acc_sh.at[pl.ds(s * RPT, RPT)])
    one = jnp.full((L,), 1.0, dtype=jnp.float32)

    def fill(i, carry):
        ones_v[i, :] = one
        return carry

    lax.fori_loop(0, K, fill, 0)
    plsc.subcore_barrier()

    def chunk(jc, carry):
        pltpu.sync_copy(ones_v, acc_sh.at[idx_v.at[jc]], add=True)
        return carry

    lax.fori_loop(0, CH1, chunk, 0)
    plsc.subcore_barrier()
    pltpu.sync_copy(acc_sh.at[pl.ds(s * RPT, RPT)],
                    out_hbm.at[c, pl.ds(s * RPT, RPT)])


_deg_kernel = functools.partial(
    pl.kernel,
    out_type=jax.ShapeDtypeStruct((NC, NPAD, DEGW), jnp.float32),
    mesh=_MESH,
    scratch_types=[
        pltpu.VMEM((CH1, K), jnp.int32),
        pltpu.VMEM((K, DEGW), jnp.float32),
        pltpu.VMEM_SHARED((NPAD, DEGW), jnp.float32),
    ],
    compiler_params=_SC_PARAMS,
)(_deg_body)


def _edge_body(src_hbm, dst_hbm, y2_hbm, zy_hbm, out_hbm,
               srcv, dstv, bufs, acc_sh, gsem, ssem):
    c = lax.axis_index("c")
    s = lax.axis_index("s")
    pltpu.sync_copy(src_hbm.at[c, s], srcv)
    pltpu.sync_copy(dst_hbm.at[s], dstv)
    pltpu.sync_copy(zy_hbm.at[pl.ds(s * RPT, RPT)],
                    acc_sh.at[pl.ds(s * RPT, RPT)])
    plsc.subcore_barrier()
    ytab = y2_hbm

    def gather(jc, slot):
        return pltpu.make_async_copy(ytab.at[srcv.at[jc]], bufs.at[slot],
                                     gsem.at[slot])

    def scatter(jc, slot):
        return pltpu.make_async_copy(bufs.at[slot], acc_sh.at[dstv.at[jc]],
                                     ssem.at[slot])

    for j in range(GRP):
        gather(j, j).start()

    def group(g, carry):
        base = g * GRP
        for b in (0, 1, 2):
            gather(base + b, b).wait()
            scatter(base + b, b).start(add=True)
        for b in (3, 4, 5):
            gather(base + b, b).wait()
            scatter(base + b, b).start(add=True)
        for b in (0, 1, 2):
            scatter(base + b, b).wait()

            @pl.when(base + GRP + b < NGRP * GRP)
            def _(b=b):
                gather(base + GRP + b, b).start()
        for b in (3, 4, 5):
            scatter(base + b, b).wait()

            @pl.when(base + GRP + b < NGRP * GRP)
            def _(b=b):
                gather(base + GRP + b, b).start()
        return carry

    lax.fori_loop(0, NGRP, group, 0)
    for jc in range(NGRP * GRP, CH2):
        gather(jc, 0).start()
        gather(jc, 0).wait()
        scatter(jc, 0).start(add=True)
        scatter(jc, 0).wait()
    plsc.subcore_barrier()
    pltpu.sync_copy(acc_sh.at[pl.ds(s * RPT, RPT)],
                    out_hbm.at[c, pl.ds(s * RPT, RPT)])


_edge_kernel = functools.partial(
    pl.kernel,
    out_type=jax.ShapeDtypeStruct((NC, NPAD, DH), jnp.float32),
    mesh=_MESH,
    scratch_types=[
        pltpu.VMEM((CH2, KE), jnp.int32),
        pltpu.VMEM((CH2, KE), jnp.int32),
        pltpu.VMEM((GRP, KE, DH), jnp.float32),
        pltpu.VMEM_SHARED((NPAD, DH), jnp.float32),
        pltpu.SemaphoreType.DMA((GRP,)),
        pltpu.SemaphoreType.DMA((GRP,)),
    ],
    compiler_params=_SC_PARAMS,
)(_edge_body)


BR2 = 2000


def _mm_body(x_ref, w_ref, dg_ref, y_ref):
    deg = dg_ref[0, :, 0:1] + dg_ref[1, :, 0:1] + 1.0
    dinv = lax.rsqrt(deg)
    xw = jnp.dot(x_ref[...], w_ref[...], preferred_element_type=jnp.float32)
    y_ref[...] = xw * dinv


def _mm_kernel(x, w, dega):
    return pl.pallas_call(
        _mm_body,
        grid=(N // BR2,),
        in_specs=[
            pl.BlockSpec((BR2, D), lambda i: (i, 0)),
            pl.BlockSpec((D, D), lambda i: (0, 0)),
            pl.BlockSpec((NC, BR2, DEGW), lambda i: (0, i, 0)),
        ],
        out_specs=pl.BlockSpec((BR2, D), lambda i: (i, 0)),
        out_shape=jax.ShapeDtypeStruct((N, D), jnp.float32),
    )(x, w, dega)


BR4 = 1000


def _ep_body(acc_ref, y_ref, dg_ref, b_ref, o_ref):
    deg = dg_ref[0, :, 0:1] + dg_ref[1, :, 0:1] + 1.0
    dinv = lax.rsqrt(deg)
    h = jnp.concatenate([acc_ref[0], acc_ref[1]], axis=1) + y_ref[...]
    o_ref[...] = h * dinv + b_ref[...]


def _ep_kernel(acc, y, dega, b2):
    return pl.pallas_call(
        _ep_body,
        grid=(N // BR4,),
        in_specs=[
            pl.BlockSpec((NC, BR4, DH), lambda i: (0, i, 0)),
            pl.BlockSpec((BR4, D), lambda i: (i, 0)),
            pl.BlockSpec((NC, BR4, DEGW), lambda i: (0, i, 0)),
            pl.BlockSpec((1, D), lambda i: (0, 0)),
        ],
        out_specs=pl.BlockSpec((BR4, D), lambda i: (i, 0)),
        out_shape=jax.ShapeDtypeStruct((N, D), jnp.float32),
    )(acc, y, dega, b2)


def kernel(x, edge_index, W, b):
    src = edge_index[0]
    dst = edge_index[1]
    pad1 = N + jnp.arange(EPAD1 - EPT1, dtype=jnp.int32) % (NPAD - N)
    pad2 = N + jnp.arange(EPAD2 - EPT2, dtype=jnp.int32) % (NPAD - N)
    dst_t1 = jnp.concatenate(
        [dst.reshape(NW, EPT1),
         jnp.broadcast_to(pad1, (NW, EPAD1 - EPT1))],
        axis=1).reshape(NW, CH1, K)
    src2 = src * 2
    src_t2 = jnp.pad(jnp.stack([src2, src2 + 1]).reshape(NC, NS, EPT2),
                     ((0, 0), (0, 0), (0, EPAD2 - EPT2)),
                     constant_values=0).reshape(NC, NS, CH2, KE)
    dst_t2 = jnp.concatenate(
        [dst.reshape(NS, EPT2),
         jnp.broadcast_to(pad2, (NS, EPAD2 - EPT2))],
        axis=1).reshape(NS, CH2, KE)
    zeros_deg = jnp.zeros((NPAD, DEGW), jnp.float32)
    zeros_y = jnp.zeros((NPAD, DH), jnp.float32)

    dega = _deg_kernel(dst_t1, zeros_deg)
    y = _mm_kernel(x, W, dega)
    acc = _edge_kernel(src_t2, dst_t2, y.reshape(2 * N, DH), zeros_y)
    return _ep_kernel(acc, y, dega, b.reshape(1, D))


# speedup vs baseline: 1.0644x; 1.0539x over previous
import functools

import jax
import jax.numpy as jnp
from jax import lax
from jax.experimental import pallas as pl
from jax.experimental.pallas import tpu as pltpu
from jax.experimental.pallas import tpu_sc as plsc

N = 10000
E = 320000
D = 128
DH = D // 2

NC = 2
NS = 16
NW = NC * NS
L = 16

NPAD = 10240
RPT = NPAD // NS
K = 128
DEGW = 16

EPT1 = E // NW
CH1 = 80
EPAD1 = CH1 * K

EPT2 = E // NS
KE = 128
CH2 = 157
EPAD2 = CH2 * KE

GRP = 6
NGRP = CH2 // GRP

_MESH = plsc.VectorSubcoreMesh(core_axis_name="c", subcore_axis_name="s",
                               num_cores=NC, num_subcores=NS)
_SC_PARAMS = pltpu.CompilerParams(use_tc_tiling_on_sc=False)


def _deg_body(dst_hbm, zdeg_hbm, out_hbm, idx_v, ones_v, acc_sh):
    c = lax.axis_index("c")
    s = lax.axis_index("s")
    w = c * NS + s
    pltpu.sync_copy(dst_hbm.at[w], idx_v)
    pltpu.sync_copy(zdeg_hbm.at[pl.ds(s * RPT, RPT)],
                    acc_sh.at[pl.ds(s * RPT, RPT)])
    one = jnp.full((L,), 1.0, dtype=jnp.float32)

    def fill(i, carry):
        ones_v[i, :] = one
        return carry

    lax.fori_loop(0, K, fill, 0)
    plsc.subcore_barrier()

    def chunk(jc, carry):
        pltpu.sync_copy(ones_v, acc_sh.at[idx_v.at[jc]], add=True)
        return carry

    lax.fori_loop(0, CH1, chunk, 0)
    plsc.subcore_barrier()
    pltpu.sync_copy(acc_sh.at[pl.ds(s * RPT, RPT)],
                    out_hbm.at[c, pl.ds(s * RPT, RPT)])


_deg_kernel = functools.partial(
    pl.kernel,
    out_type=jax.ShapeDtypeStruct((NC, NPAD, DEGW), jnp.float32),
    mesh=_MESH,
    scratch_types=[
        pltpu.VMEM((CH1, K), jnp.int32),
        pltpu.VMEM((K, DEGW), jnp.float32),
        pltpu.VMEM_SHARED((NPAD, DEGW), jnp.float32),
    ],
    compiler_params=_SC_PARAMS,
)(_deg_body)


def _edge_body(src_hbm, dst_hbm, y2_hbm, zy_hbm, out_hbm,
               srcv, dstv, bufs, acc_sh, gsem, ssem):
    c = lax.axis_index("c")
    s = lax.axis_index("s")
    pltpu.sync_copy(src_hbm.at[s], srcv)
    pltpu.sync_copy(dst_hbm.at[s], dstv)
    pltpu.sync_copy(zy_hbm.at[pl.ds(s * RPT, RPT)],
                    acc_sh.at[pl.ds(s * RPT, RPT)])
    plsc.subcore_barrier()
    ytab = y2_hbm.at[c]

    def gather(jc, slot):
        return pltpu.make_async_copy(ytab.at[srcv.at[jc]], bufs.at[slot],
                                     gsem.at[slot])

    def scatter(jc, slot):
        return pltpu.make_async_copy(bufs.at[slot], acc_sh.at[dstv.at[jc]],
                                     ssem.at[slot])

    for j in range(GRP):
        gather(j, j).start()

    def group(g, carry):
        base = g * GRP
        for b in (0, 1, 2):
            gather(base + b, b).wait()
            scatter(base + b, b).start(add=True)
        for b in (3, 4, 5):
            gather(base + b, b).wait()
            scatter(base + b, b).start(add=True)
        for b in (0, 1, 2):
            scatter(base + b, b).wait()

            @pl.when(base + GRP + b < NGRP * GRP)
            def _(b=b):
                gather(base + GRP + b, b).start()
        for b in (3, 4, 5):
            scatter(base + b, b).wait()

            @pl.when(base + GRP + b < NGRP * GRP)
            def _(b=b):
                gather(base + GRP + b, b).start()
        return carry

    lax.fori_loop(0, NGRP, group, 0)
    for jc in range(NGRP * GRP, CH2):
        gather(jc, 0).start()
        gather(jc, 0).wait()
        scatter(jc, 0).start(add=True)
        scatter(jc, 0).wait()
    plsc.subcore_barrier()
    pltpu.sync_copy(acc_sh.at[pl.ds(s * RPT, RPT)],
                    out_hbm.at[c, pl.ds(s * RPT, RPT)])


_edge_kernel = functools.partial(
    pl.kernel,
    out_type=jax.ShapeDtypeStruct((NC, NPAD, DH), jnp.float32),
    mesh=_MESH,
    scratch_types=[
        pltpu.VMEM((CH2, KE), jnp.int32),
        pltpu.VMEM((CH2, KE), jnp.int32),
        pltpu.VMEM((GRP, KE, DH), jnp.float32),
        pltpu.VMEM_SHARED((NPAD, DH), jnp.float32),
        pltpu.SemaphoreType.DMA((GRP,)),
        pltpu.SemaphoreType.DMA((GRP,)),
    ],
    compiler_params=_SC_PARAMS,
)(_edge_body)


BR2 = 2000


def _mm_body(x_ref, w_ref, dg_ref, y_ref):
    deg = dg_ref[0, :, 0:1] + dg_ref[1, :, 0:1] + 1.0
    dinv = lax.rsqrt(deg)
    xw = jnp.dot(x_ref[...], w_ref[...], preferred_element_type=jnp.float32)
    y_ref[...] = xw * dinv


def _mm_kernel(x, w, dega):
    return pl.pallas_call(
        _mm_body,
        grid=(N // BR2,),
        in_specs=[
            pl.BlockSpec((BR2, D), lambda i: (i, 0)),
            pl.BlockSpec((D, D), lambda i: (0, 0)),
            pl.BlockSpec((NC, BR2, DEGW), lambda i: (0, i, 0)),
        ],
        out_specs=pl.BlockSpec((BR2, D), lambda i: (i, 0)),
        out_shape=jax.ShapeDtypeStruct((N, D), jnp.float32),
    )(x, w, dega)


BR4 = 1000


def _ep_body(acc_ref, y_ref, dg_ref, b_ref, o_ref):
    deg = dg_ref[0, :, 0:1] + dg_ref[1, :, 0:1] + 1.0
    dinv = lax.rsqrt(deg)
    h = jnp.concatenate([acc_ref[0], acc_ref[1]], axis=1) + y_ref[...]
    o_ref[...] = h * dinv + b_ref[...]


def _ep_kernel(acc, y, dega, b2):
    return pl.pallas_call(
        _ep_body,
        grid=(N // BR4,),
        in_specs=[
            pl.BlockSpec((NC, BR4, DH), lambda i: (0, i, 0)),
            pl.BlockSpec((BR4, D), lambda i: (i, 0)),
            pl.BlockSpec((NC, BR4, DEGW), lambda i: (0, i, 0)),
            pl.BlockSpec((1, D), lambda i: (0, 0)),
        ],
        out_specs=pl.BlockSpec((BR4, D), lambda i: (i, 0)),
        out_shape=jax.ShapeDtypeStruct((N, D), jnp.float32),
    )(acc, y, dega, b2)


def kernel(x, edge_index, W, b):
    src = edge_index[0]
    dst = edge_index[1]
    pad1 = N + jnp.arange(EPAD1 - EPT1, dtype=jnp.int32) % (NPAD - N)
    pad2 = N + jnp.arange(EPAD2 - EPT2, dtype=jnp.int32) % (NPAD - N)
    dst_t1 = jnp.concatenate(
        [dst.reshape(NW, EPT1),
         jnp.broadcast_to(pad1, (NW, EPAD1 - EPT1))],
        axis=1).reshape(NW, CH1, K)
    src_t2 = jnp.pad(src.reshape(NS, EPT2), ((0, 0), (0, EPAD2 - EPT2)),
                     constant_values=0).reshape(NS, CH2, KE)
    dst_t2 = jnp.concatenate(
        [dst.reshape(NS, EPT2),
         jnp.broadcast_to(pad2, (NS, EPAD2 - EPT2))],
        axis=1).reshape(NS, CH2, KE)
    zeros_deg = jnp.zeros((NPAD, DEGW), jnp.float32)
    zeros_y = jnp.zeros((NPAD, DH), jnp.float32)

    dega = _deg_kernel(dst_t1, zeros_deg)
    y = _mm_kernel(x, W, dega)
    y2 = jnp.stack([y[:, :DH], y[:, DH:]])
    acc = _edge_kernel(src_t2, dst_t2, y2, zeros_y)
    return _ep_kernel(acc, y, dega, b.reshape(1, D))


# speedup vs baseline: 1.0788x; 1.0135x over previous
import functools

import jax
import jax.numpy as jnp
from jax import lax
from jax.experimental import pallas as pl
from jax.experimental.pallas import tpu as pltpu
from jax.experimental.pallas import tpu_sc as plsc

N = 10000
E = 320000
D = 128
DH = D // 2

NC = 2
NS = 16
NW = NC * NS
L = 16

NPAD = 10240
RPT = NPAD // NS
K = 128
DEGW = 16

EPT1 = E // NW
CH1 = 80
EPAD1 = CH1 * K

EPT2 = E // NS
KE = 128
CH2 = 157
EPAD2 = CH2 * KE

GRP = 6
NGRP = CH2 // GRP

_MESH = plsc.VectorSubcoreMesh(core_axis_name="c", subcore_axis_name="s",
                               num_cores=NC, num_subcores=NS)
_SC_PARAMS = pltpu.CompilerParams(use_tc_tiling_on_sc=False)


def _deg_body(dst_hbm, zdeg_hbm, out_hbm, idx_v, ones_v, acc_sh, dsem):
    c = lax.axis_index("c")
    s = lax.axis_index("s")
    w = c * NS + s
    pltpu.sync_copy(dst_hbm.at[w], idx_v)
    pltpu.sync_copy(zdeg_hbm.at[pl.ds(s * RPT, RPT)],
                    acc_sh.at[pl.ds(s * RPT, RPT)])
    one = jnp.full((L,), 1.0, dtype=jnp.float32)

    def fill(i, carry):
        ones_v[i, :] = one
        return carry

    lax.fori_loop(0, K, fill, 0)
    plsc.subcore_barrier()

    def dchunk(g, carry):
        base = g * 8
        for j in range(8):
            pltpu.async_copy(ones_v, acc_sh.at[idx_v.at[base + j]], dsem,
                             add=True)
        for j in range(8):
            pltpu.make_async_copy(ones_v, acc_sh.at[idx_v.at[base + j]],
                                  dsem).wait()
        return carry

    lax.fori_loop(0, CH1 // 8, dchunk, 0)
    plsc.subcore_barrier()
    pltpu.sync_copy(acc_sh.at[pl.ds(s * RPT, RPT)],
                    out_hbm.at[c, pl.ds(s * RPT, RPT)])


_deg_kernel = functools.partial(
    pl.kernel,
    out_type=jax.ShapeDtypeStruct((NC, NPAD, DEGW), jnp.float32),
    mesh=_MESH,
    scratch_types=[
        pltpu.VMEM((CH1, K), jnp.int32),
        pltpu.VMEM((K, DEGW), jnp.float32),
        pltpu.VMEM_SHARED((NPAD, DEGW), jnp.float32),
        pltpu.SemaphoreType.DMA,
    ],
    compiler_params=_SC_PARAMS,
)(_deg_body)


def _edge_body(src_hbm, dst_hbm, y2_hbm, zy_hbm, out_hbm,
               srcv, dstv, bufs, acc_sh, gsem, ssem):
    c = lax.axis_index("c")
    s = lax.axis_index("s")
    pltpu.sync_copy(src_hbm.at[s], srcv)
    pltpu.sync_copy(dst_hbm.at[s], dstv)
    pltpu.sync_copy(zy_hbm.at[pl.ds(s * RPT, RPT)],
                    acc_sh.at[pl.ds(s * RPT, RPT)])
    plsc.subcore_barrier()
    ytab = y2_hbm.at[c]

    def gather(jc, slot):
        return pltpu.make_async_copy(ytab.at[srcv.at[jc]], bufs.at[slot],
                                     gsem.at[slot])

    def scatter(jc, slot):
        return pltpu.make_async_copy(bufs.at[slot], acc_sh.at[dstv.at[jc]],
                                     ssem.at[slot])

    for j in range(GRP):
        gather(j, j).start()

    def group(g, carry):
        base = g * GRP
        for b in (0, 1, 2):
            gather(base + b, b).wait()
            scatter(base + b, b).start(add=True)
        for b in (3, 4, 5):
            gather(base + b, b).wait()
            scatter(base + b, b).start(add=True)
        for b in (0, 1, 2):
            scatter(base + b, b).wait()

            @pl.when(base + GRP + b < NGRP * GRP)
            def _(b=b):
                gather(base + GRP + b, b).start()
        for b in (3, 4, 5):
            scatter(base + b, b).wait()

            @pl.when(base + GRP + b < NGRP * GRP)
            def _(b=b):
                gather(base + GRP + b, b).start()
        return carry

    lax.fori_loop(0, NGRP, group, 0)
    for jc in range(NGRP * GRP, CH2):
        gather(jc, 0).start()
        gather(jc, 0).wait()
        scatter(jc, 0).start(add=True)
        scatter(jc, 0).wait()
    plsc.subcore_barrier()
    pltpu.sync_copy(acc_sh.at[pl.ds(s * RPT, RPT)],
                    out_hbm.at[c, pl.ds(s * RPT, RPT)])


_edge_kernel = functools.partial(
    pl.kernel,
    out_type=jax.ShapeDtypeStruct((NC, NPAD, DH), jnp.float32),
    mesh=_MESH,
    scratch_types=[
        pltpu.VMEM((CH2, KE), jnp.int32),
        pltpu.VMEM((CH2, KE), jnp.int32),
        pltpu.VMEM((GRP, KE, DH), jnp.float32),
        pltpu.VMEM_SHARED((NPAD, DH), jnp.float32),
        pltpu.SemaphoreType.DMA((GRP,)),
        pltpu.SemaphoreType.DMA((GRP,)),
    ],
    compiler_params=_SC_PARAMS,
)(_edge_body)


BR2 = 2000


def _mm_body(x_ref, w_ref, dg_ref, y_ref):
    deg = dg_ref[0, :, 0:1] + dg_ref[1, :, 0:1] + 1.0
    dinv = lax.rsqrt(deg)
    xw = jnp.dot(x_ref[...], w_ref[...], preferred_element_type=jnp.float32)
    y_ref[...] = xw * dinv


def _mm_kernel(x, w, dega):
    return pl.pallas_call(
        _mm_body,
        grid=(N // BR2,),
        in_specs=[
            pl.BlockSpec((BR2, D), lambda i: (i, 0)),
            pl.BlockSpec((D, D), lambda i: (0, 0)),
            pl.BlockSpec((NC, BR2, DEGW), lambda i: (0, i, 0)),
        ],
        out_specs=pl.BlockSpec((BR2, D), lambda i: (i, 0)),
        out_shape=jax.ShapeDtypeStruct((N, D), jnp.float32),
    )(x, w, dega)


BR4 = 1000


def _ep_body(acc_ref, y_ref, dg_ref, b_ref, o_ref):
    deg = dg_ref[0, :, 0:1] + dg_ref[1, :, 0:1] + 1.0
    dinv = lax.rsqrt(deg)
    h = jnp.concatenate([acc_ref[0], acc_ref[1]], axis=1) + y_ref[...]
    o_ref[...] = h * dinv + b_ref[...]


def _ep_kernel(acc, y, dega, b2):
    return pl.pallas_call(
        _ep_body,
        grid=(N // BR4,),
        in_specs=[
            pl.BlockSpec((NC, BR4, DH), lambda i: (0, i, 0)),
            pl.BlockSpec((BR4, D), lambda i: (i, 0)),
            pl.BlockSpec((NC, BR4, DEGW), lambda i: (0, i, 0)),
            pl.BlockSpec((1, D), lambda i: (0, 0)),
        ],
        out_specs=pl.BlockSpec((BR4, D), lambda i: (i, 0)),
        out_shape=jax.ShapeDtypeStruct((N, D), jnp.float32),
    )(acc, y, dega, b2)


def kernel(x, edge_index, W, b):
    src = edge_index[0]
    dst = edge_index[1]
    pad1 = N + jnp.arange(EPAD1 - EPT1, dtype=jnp.int32) % (NPAD - N)
    pad2 = N + jnp.arange(EPAD2 - EPT2, dtype=jnp.int32) % (NPAD - N)
    dst_t1 = jnp.concatenate(
        [dst.reshape(NW, EPT1),
         jnp.broadcast_to(pad1, (NW, EPAD1 - EPT1))],
        axis=1).reshape(NW, CH1, K)
    src_t2 = jnp.pad(src.reshape(NS, EPT2), ((0, 0), (0, EPAD2 - EPT2)),
                     constant_values=0).reshape(NS, CH2, KE)
    dst_t2 = jnp.concatenate(
        [dst.reshape(NS, EPT2),
         jnp.broadcast_to(pad2, (NS, EPAD2 - EPT2))],
        axis=1).reshape(NS, CH2, KE)
    zeros_deg = jnp.zeros((NPAD, DEGW), jnp.float32)
    zeros_y = jnp.zeros((NPAD, DH), jnp.float32)

    dega = _deg_kernel(dst_t1, zeros_deg)
    y = _mm_kernel(x, W, dega)
    y2 = jnp.stack([y[:, :DH], y[:, DH:]])
    acc = _edge_kernel(src_t2, dst_t2, y2, zeros_y)
    return _ep_kernel(acc, y, dega, b.reshape(1, D))
